# Initial kernel scaffold; baseline (speedup 1.0000x reference)
import jax
import jax.numpy as jnp
from jax.experimental import pallas as pl


def kernel(seizure_logits, predict_logits, preictal_logits, predicts_time, labels):
    raise NotImplementedError("write your pallas kernel here")


# speedup vs baseline: 9.7266x; 9.7266x over previous
import functools

import jax
import jax.numpy as jnp
from jax import lax
from jax.experimental import pallas as pl
from jax.experimental.pallas import tpu as pltpu
from jax.experimental.pallas import tpu_sc as plsc

_B = 131072
_T = 50
_NC = 2
_NS = 16
_L = 16
_NW = _NC * _NS
_ROWS_PER_W = _B // _NW
_CHUNK = 512
_NCHUNK = _ROWS_PER_W // _CHUNK
_GROUPS = _CHUNK // _L


def _sc_mode_hist(pt_hbm, lab_hbm, modes_hbm, ratio_hbm, hist_hbm,
                  rows_v, lab_v, modes_v, ratio_v, hist2_v, hist_v):
    wid = lax.axis_index("s") * _NC + lax.axis_index("c")
    iota = lax.iota(jnp.int32, _L)
    zeros = jnp.zeros((_L,), jnp.int32)
    ones = jnp.full((_L,), 1, jnp.int32)
    neg1 = jnp.full((_L,), -1, jnp.int32)
    lane_row = iota * 32

    for k in range(_L * 32 // _L):
        hist2_v[pl.ds(k * _L, _L)] = zeros

    base_row = wid * _ROWS_PER_W

    iota_t = iota * _T

    def group_body(g, carry):
        row0 = g * _L
        base = iota_t + g * (_L * _T)
        acc = zeros
        for t in range(_T):
            v = plsc.load_gather(rows_v, [base + t])
            acc = acc + (jnp.int32(1) << (v * 6))
        cs = [(acc >> (6 * c)) & 63 for c in range(5)]
        best = cs[4]
        mode = jnp.full((_L,), 4, jnp.int32)
        for c in (3, 2, 1, 0):
            ge = cs[c] >= best
            best = jnp.where(ge, cs[c], best)
            mode = jnp.where(ge, jnp.full((_L,), c, jnp.int32), mode)
        m2 = neg1
        for c in range(5):
            m2 = jnp.maximum(m2, jnp.where(mode == jnp.full((_L,), c, jnp.int32), neg1, cs[c]))
        ratio = m2.astype(jnp.float32) / best.astype(jnp.float32)
        modes_v[pl.ds(row0, _L)] = mode
        ratio_v[pl.ds(row0, _L)] = ratio
        lab = lab_v[pl.ds(row0, _L)]
        plsc.addupdate_scatter(hist2_v, [lane_row + lab * 5 + mode], ones)
        return carry

    for ch in range(_NCHUNK):
        row0 = base_row + ch * _CHUNK
        pltpu.sync_copy(pt_hbm.at[pl.ds(row0 * _T, _CHUNK * _T)], rows_v)
        pltpu.sync_copy(lab_hbm.at[pl.ds(row0, _CHUNK)], lab_v)
        lax.fori_loop(0, _GROUPS, group_body, 0)
        pltpu.sync_copy(modes_v, modes_hbm.at[pl.ds(row0, _CHUNK)])
        pltpu.sync_copy(ratio_v, ratio_hbm.at[pl.ds(row0, _CHUNK)])

    lo = zeros
    hi = zeros
    for r in range(_L):
        lo = lo + hist2_v[pl.ds(r * 32, _L)]
        hi = hi + hist2_v[pl.ds(r * 32 + _L, _L)]
    hist_v[pl.ds(0, _L)] = lo
    hist_v[pl.ds(_L, _L)] = hi
    pltpu.sync_copy(hist_v, hist_hbm.at[wid])


def _make_sc_call():
    mesh = plsc.VectorSubcoreMesh(core_axis_name="c", subcore_axis_name="s")
    return pl.kernel(
        _sc_mode_hist,
        mesh=mesh,
        compiler_params=pltpu.CompilerParams(needs_layout_passes=False),
        out_type=[
            jax.ShapeDtypeStruct((_B,), jnp.int32),
            jax.ShapeDtypeStruct((_B,), jnp.float32),
            jax.ShapeDtypeStruct((_NW, 32), jnp.int32),
        ],
        scratch_types=[
            pltpu.VMEM((_CHUNK * _T,), jnp.int32),
            pltpu.VMEM((_CHUNK,), jnp.int32),
            pltpu.VMEM((_CHUNK,), jnp.int32),
            pltpu.VMEM((_CHUNK,), jnp.float32),
            pltpu.VMEM((_L * 32,), jnp.int32),
            pltpu.VMEM((32,), jnp.int32),
        ],
    )


def _ce_loss_body(s0, s1, p0, p1, q0, q1, q2, lab_ref, out_ref):
    lab = lab_ref[...]
    a0 = s0[...]
    a1 = s1[...]
    m = jnp.maximum(a0, a1)
    lse_s = m + jnp.log1p(jnp.exp(jnp.minimum(a0, a1) - m))
    term = jnp.where(lab == 4, a1, a0) - lse_s

    b0 = p0[...]
    b1 = p1[...]
    m = jnp.maximum(b0, b1)
    lse_p = m + jnp.log1p(jnp.exp(jnp.minimum(b0, b1) - m))
    term = term + jnp.where(lab == 0, b0, jnp.where(lab <= 3, b1, 0.0))
    term = term - (lab <= 3).astype(jnp.float32) * lse_p

    c0 = q0[...]
    c1 = q1[...]
    c2 = q2[...]
    t0 = jnp.where(lab == 1, 0.9, 0.0) + jnp.where(lab == 2, 0.05, 0.0)
    t1 = (jnp.where(lab == 1, 0.1, 0.0) + jnp.where(lab == 2, 0.9, 0.0)
          + jnp.where(lab == 3, 0.1, 0.0))
    t2 = jnp.where(lab == 2, 0.05, 0.0) + jnp.where(lab == 3, 0.9, 0.0)
    m = jnp.maximum(jnp.maximum(c0, c1), c2)
    lse_q = m + jnp.log(jnp.exp(c0 - m) + jnp.exp(c1 - m) + jnp.exp(c2 - m))
    sumt = ((lab >= 1) & (lab <= 3)).astype(jnp.float32)
    term = term + (t0 * c0 + t1 * c1 + t2 * c2 - sumt * lse_q)

    out_ref[...] = (-jnp.sum(term) / _B).reshape(1, 1)


def _metric_scalars(TP, FN, FP, TN):
    eps = jnp.float32(1e-06)
    accuracy = (TP + TN) / (TP + FN + FP + TN + eps)
    specificity = TN / (TN + FP + eps)
    sensitivity = TP / (TP + FN + eps)
    precision = TP / (TP + FP + eps)
    F1_score = 2 * TP / (2 * TP + FP + FN + eps)
    return accuracy, specificity, sensitivity, precision, F1_score


def kernel(seizure_logits, predict_logits, preictal_logits, predicts_time, labels):
    labels = labels.astype(jnp.int32)
    pt = predicts_time.astype(jnp.int32)

    modes, ratio_top, hist_parts = _make_sc_call()(pt.reshape(-1), labels)

    shape2d = (_B // 256, 256)
    cols = [seizure_logits[:, 0], seizure_logits[:, 1],
            predict_logits[:, 0], predict_logits[:, 1],
            preictal_logits[:, 0], preictal_logits[:, 1], preictal_logits[:, 2]]
    cols = [c.reshape(shape2d) for c in cols]
    lab2d = labels.reshape(shape2d)
    loss = pl.pallas_call(
        _ce_loss_body,
        out_shape=jax.ShapeDtypeStruct((1, 1), jnp.float32),
    )(*cols, lab2d)[0, 0]

    cm5 = hist_parts.sum(axis=0)[:25].reshape(5, 5)

    bs = _B
    err = cm5.astype(jnp.float32) / bs
    accuracy_five = jnp.trace(err)
    TP2s = err[:4, :4].sum(); FN2s = err[:4, 4].sum(); FP2s = err[4, :4].sum(); TN2s = err[4, 4]
    acc2s, spec2s, sens2s, prec2s, f12s = _metric_scalars(TP2s, FN2s, FP2s, TN2s)
    TP2p = err[0, 0]; FN2p = err[0, 1:4].sum(); FP2p = err[1:4, 0].sum(); TN2p = err[1:4, 1:4].sum()
    acc2p, spec2p, sens2p, prec2p, f12p = _metric_scalars(TP2p, FN2p, FP2p, TN2p)
    accuracy_three = err[0, 0] + err[1:4, 1:4].sum() + err[4, 4]
    tiny = jnp.float32(1e-06)
    accuracy_ictal = err[4, 4] / (err[4, :].sum() + tiny)
    accuracy_preictal = err[1:4, 1:4].sum() / (err[1:4, :].sum() + tiny)
    accuracy_preictalI = err[3, 3] / (err[3, :].sum() + tiny)
    accuracy_preictalII = err[2, 2] / (err[2, :].sum() + tiny)
    accuracy_preictalIII = err[1, 1] / (err[1, :].sum() + tiny)
    accuracy_interictal = err[0, 0] / (err[0, :].sum() + tiny)
    cm3 = jnp.stack([
        jnp.stack([cm5[0, 0], cm5[0, 1:4].sum(), cm5[0, 4]]),
        jnp.stack([cm5[1:4, 0].sum(), cm5[1:4, 1:4].sum(), cm5[1:4, 4].sum()]),
        jnp.stack([cm5[4, 0], cm5[4, 1:4].sum(), cm5[4, 4]]),
    ])
    cm2p = jnp.stack([
        jnp.stack([cm5[0, 0], cm5[0, 1:4].sum()]),
        jnp.stack([cm5[1:4, 0].sum(), cm5[1:4, 1:4].sum()]),
    ])
    cm2s = jnp.stack([
        jnp.stack([cm5[:4, :4].sum(), cm5[:4, 4].sum()]),
        jnp.stack([cm5[4, :4].sum(), cm5[4, 4]]),
    ])
    return (loss, modes, ratio_top, accuracy_five, acc2s, acc2p, accuracy_three,
            accuracy_ictal, accuracy_preictal, accuracy_preictalI,
            accuracy_preictalII, accuracy_preictalIII, accuracy_interictal,
            cm2s, cm2p, cm3, cm5, spec2s, sens2s, prec2s, f12s,
            spec2p, sens2p, prec2p, f12p)
